# baseline (speedup 1.0000x reference)
import jax
import jax.numpy as jnp
from jax import lax
from jax.experimental import pallas as pl
from jax.experimental.pallas import tpu as pltpu
from jax.experimental.pallas import tpu_sc as plsc

B = 4096
L = 50
D = 128
VOCAB_ROWS = 100000
NC, NS = 2, 16
NW = NC * NS
NHALF = 2
BH = B // NHALF
SEG = 2 * BH
SEG_PER_W = SEG // NW
CH = 2
CHROWS = CH * L
CHPAD = 104
NCHUNK = SEG_PER_W // CH
DW = D // 2
NB = DW // 16
KBUF = 4


def _pack_body(x_ref, out_ref):
  xb = lax.bitcast_convert_type(x_ref[...], jnp.int32)
  rnd = lax.shift_right_logical(xb, 16) & 1
  bits = lax.shift_right_logical(xb + 0x7FFF + rnd, 16)
  lo = bits[:, :DW]
  hi = bits[:, DW:]
  out_ref[...] = lo | lax.shift_left(hi, 16)


def _pack_table(emb):
  rb = 2000
  return pl.pallas_call(
      _pack_body,
      grid=(VOCAB_ROWS // rb,),
      in_specs=[pl.BlockSpec((rb, D), lambda i: (i, 0))],
      out_specs=pl.BlockSpec((rb, DW), lambda i: (i, 0)),
      out_shape=jax.ShapeDtypeStruct((VOCAB_ROWS, DW), jnp.int32),
  )(emb)


def _sc_pool_body(emb_hbm, idx_hbm, out_hbm, idx_v, buf0, buf1, buf2, buf3,
                  res, sem0, sem1, sem2, sem3):
  c = lax.axis_index("c")
  s = lax.axis_index("s")
  wid = s * NC + c
  bufs = (buf0, buf1, buf2, buf3)
  sems = (sem0, sem1, sem2, sem3)

  pltpu.sync_copy(idx_hbm.at[wid], idx_v)

  hi_mask = jnp.full((16,), -65536, jnp.int32)

  def bf2x(words):
    a = lax.bitcast_convert_type(lax.shift_left(words, 16), jnp.float32)
    b = lax.bitcast_convert_type(lax.bitwise_and(words, hi_mask),
                                 jnp.float32)
    return a, b

  def reduce_chunk(buf, cbase):
    for sg in range(CH):
      acc_a = [None] * NB
      acc_b = [None] * NB
      for r in range(L):
        for d in range(NB):
          a, b = bf2x(buf[sg * L + r, pl.ds(16 * d, 16)])
          if r == 0:
            acc_a[d], acc_b[d] = a, b
          else:
            acc_a[d] = acc_a[d] + a
            acc_b[d] = acc_b[d] + b
      for d in range(NB):
        res[cbase + sg, pl.ds(32 * d, 16)] = acc_a[d]
        res[cbase + sg, pl.ds(32 * d + 16, 16)] = acc_b[d]

  for j in range(KBUF):
    pltpu.async_copy(emb_hbm.at[idx_v.at[j]], bufs[j], sems[j])

  def body_ring(g, carry):
    c0 = KBUF * g
    for j in range(KBUF):
      cj = c0 + j
      pltpu.make_async_copy(emb_hbm.at[idx_v.at[cj]], bufs[j], sems[j]).wait()
      reduce_chunk(bufs[j], cj * CH)

      @pl.when(cj + KBUF < NCHUNK)
      def _():
        pltpu.async_copy(emb_hbm.at[idx_v.at[cj + KBUF]], bufs[j], sems[j])
    return carry

  lax.fori_loop(0, NCHUNK // KBUF, body_ring, 0)
  pltpu.sync_copy(res, out_hbm.at[pl.ds(wid * SEG_PER_W, SEG_PER_W)])


def _sc_pool(emb_words, idx):
  mesh = plsc.VectorSubcoreMesh(core_axis_name="c", subcore_axis_name="s")
  return pl.kernel(
      _sc_pool_body,
      out_type=jax.ShapeDtypeStruct((SEG, D), jnp.float32),
      mesh=mesh,
      compiler_params=pltpu.CompilerParams(use_tc_tiling_on_sc=False),
      scratch_types=[
          pltpu.VMEM((NCHUNK, CHPAD), jnp.int32),
          pltpu.VMEM((CHPAD, DW), jnp.int32),
          pltpu.VMEM((CHPAD, DW), jnp.int32),
          pltpu.VMEM((CHPAD, DW), jnp.int32),
          pltpu.VMEM((CHPAD, DW), jnp.int32),
          pltpu.VMEM((SEG_PER_W, D), jnp.float32),
          pltpu.SemaphoreType.DMA,
          pltpu.SemaphoreType.DMA,
          pltpu.SemaphoreType.DMA,
          pltpu.SemaphoreType.DMA,
      ],
  )(emb_words, idx)


def _mlp_body(x1_ref, x2_ref, w1a, w1b, b1, w2, b2, w3, b3, out_ref):
  h = jnp.dot(x1_ref[...], w1a[...], preferred_element_type=jnp.float32)
  h = h + jnp.dot(x2_ref[...], w1b[...], preferred_element_type=jnp.float32)
  h = jnp.maximum(h + b1[...], 0.0)
  h = jnp.maximum(
      jnp.dot(h, w2[...], preferred_element_type=jnp.float32) + b2[...], 0.0)
  out_ref[...] = (
      jnp.dot(h, w3[...], preferred_element_type=jnp.float32) + b3[...])


def _mlp(q, w1a, w1b, b1, w2, b2, w3p, b3p):
  bb = 512
  grid = (BH // bb,)
  h1 = w1a.shape[1]
  h2 = w2.shape[1]
  return pl.pallas_call(
      _mlp_body,
      grid=grid,
      in_specs=[
          pl.BlockSpec((bb, D), lambda i: (i, 0)),
          pl.BlockSpec((bb, D), lambda i: (i + BH // bb, 0)),
          pl.BlockSpec((D, h1), lambda i: (0, 0)),
          pl.BlockSpec((D, h1), lambda i: (0, 0)),
          pl.BlockSpec((1, h1), lambda i: (0, 0)),
          pl.BlockSpec((h1, h2), lambda i: (0, 0)),
          pl.BlockSpec((1, h2), lambda i: (0, 0)),
          pl.BlockSpec((h2, 128), lambda i: (0, 0)),
          pl.BlockSpec((1, 128), lambda i: (0, 0)),
      ],
      out_specs=pl.BlockSpec((bb, 128), lambda i: (i, 0)),
      out_shape=jax.ShapeDtypeStruct((BH, 128), jnp.float32),
  )(q, q, w1a, w1b, b1, w2, b2, w3p, b3p)


def kernel(x1, x2, emb, W1, b1, W2, b2, W3, b3):
  table = _pack_table(emb)

  perm = []
  for d in range(NB):
    perm.extend(16 * d + o for o in range(16))
    perm.extend(DW + 16 * d + o for o in range(16))
  perm = jnp.asarray(perm, jnp.int32)

  inv_l = jnp.float32(1.0 / L)
  w1a = (W1[:, :D] * inv_l).T[perm]
  w1b = (W1[:, D:] * inv_l).T[perm]
  w2 = W2.T
  w3p = jnp.zeros((W2.shape[0], 128), jnp.float32).at[:, :2].set(W3.T)
  b3p = jnp.zeros((1, 128), jnp.float32).at[0, :2].set(b3)
  b1r = b1.reshape(1, -1)
  b2r = b2.reshape(1, -1)

  outs = []
  for h in range(NHALF):
    idx = jnp.concatenate([x1[h * BH:(h + 1) * BH].reshape(-1),
                           x2[h * BH:(h + 1) * BH].reshape(-1)])
    idx = idx.reshape(NW, NCHUNK, CHROWS)
    idx = jnp.pad(idx, ((0, 0), (0, 0), (0, CHPAD - CHROWS)))
    q = _sc_pool(table, idx)
    outs.append(_mlp(q, w1a, w1b, b1r, w2, b2r, w3p, b3p)[:, :2])

  return jnp.concatenate(outs, axis=0)


# speedup vs baseline: 1.0079x; 1.0079x over previous
import jax
import jax.numpy as jnp
from jax import lax
from jax.experimental import pallas as pl
from jax.experimental.pallas import tpu as pltpu
from jax.experimental.pallas import tpu_sc as plsc

B = 4096
L = 50
D = 128
VOCAB_ROWS = 100000
NC, NS = 2, 16
NW = NC * NS
NHALF = 1
BH = B // NHALF
SEG = 2 * BH
SEG_PER_W = SEG // NW
CH = 2
CHROWS = CH * L
CHPAD = 104
NCHUNK = SEG_PER_W // CH
DW = D // 2
NB = DW // 16
KBUF = 4


def _pack_body(x_ref, out_ref):
  xb = lax.bitcast_convert_type(x_ref[...], jnp.int32)
  rnd = lax.shift_right_logical(xb, 16) & 1
  bits = lax.shift_right_logical(xb + 0x7FFF + rnd, 16)
  lo = bits[:, :DW]
  hi = bits[:, DW:]
  out_ref[...] = lo | lax.shift_left(hi, 16)


def _pack_table(emb):
  rb = 2000
  return pl.pallas_call(
      _pack_body,
      grid=(VOCAB_ROWS // rb,),
      in_specs=[pl.BlockSpec((rb, D), lambda i: (i, 0))],
      out_specs=pl.BlockSpec((rb, DW), lambda i: (i, 0)),
      out_shape=jax.ShapeDtypeStruct((VOCAB_ROWS, DW), jnp.int32),
  )(emb)


def _sc_pool_body(emb_hbm, idx_hbm, out_hbm, idx_v, buf0, buf1, buf2, buf3,
                  res, sem0, sem1, sem2, sem3):
  c = lax.axis_index("c")
  s = lax.axis_index("s")
  wid = s * NC + c
  bufs = (buf0, buf1, buf2, buf3)
  sems = (sem0, sem1, sem2, sem3)

  pltpu.sync_copy(idx_hbm.at[wid], idx_v)

  hi_mask = jnp.full((16,), -65536, jnp.int32)

  def bf2x(words):
    a = lax.bitcast_convert_type(lax.shift_left(words, 16), jnp.float32)
    b = lax.bitcast_convert_type(lax.bitwise_and(words, hi_mask),
                                 jnp.float32)
    return a, b

  def reduce_chunk(buf, cbase):
    for sg in range(CH):
      acc_a = [None] * NB
      acc_b = [None] * NB
      for r in range(L):
        for d in range(NB):
          a, b = bf2x(buf[sg * L + r, pl.ds(16 * d, 16)])
          if r == 0:
            acc_a[d], acc_b[d] = a, b
          else:
            acc_a[d] = acc_a[d] + a
            acc_b[d] = acc_b[d] + b
      for d in range(NB):
        res[cbase + sg, pl.ds(16 * d, 16)] = acc_a[d]
        res[cbase + sg, pl.ds(DW + 16 * d, 16)] = acc_b[d]

  for j in range(KBUF):
    pltpu.async_copy(emb_hbm.at[idx_v.at[j]], bufs[j], sems[j])

  def body_ring(g, carry):
    c0 = KBUF * g
    for j in range(KBUF):
      cj = c0 + j
      pltpu.make_async_copy(emb_hbm.at[idx_v.at[cj]], bufs[j], sems[j]).wait()
      reduce_chunk(bufs[j], cj * CH)

      @pl.when(cj + KBUF < NCHUNK)
      def _():
        pltpu.async_copy(emb_hbm.at[idx_v.at[cj + KBUF]], bufs[j], sems[j])
    return carry

  lax.fori_loop(0, NCHUNK // KBUF, body_ring, 0)
  pltpu.sync_copy(res, out_hbm.at[pl.ds(wid * SEG_PER_W, SEG_PER_W)])


def _sc_pool(emb_words, idx):
  mesh = plsc.VectorSubcoreMesh(core_axis_name="c", subcore_axis_name="s")
  return pl.kernel(
      _sc_pool_body,
      out_type=jax.ShapeDtypeStruct((SEG, D), jnp.float32),
      mesh=mesh,
      compiler_params=pltpu.CompilerParams(use_tc_tiling_on_sc=False),
      scratch_types=[
          pltpu.VMEM((NCHUNK, CHPAD), jnp.int32),
          pltpu.VMEM((CHPAD, DW), jnp.int32),
          pltpu.VMEM((CHPAD, DW), jnp.int32),
          pltpu.VMEM((CHPAD, DW), jnp.int32),
          pltpu.VMEM((CHPAD, DW), jnp.int32),
          pltpu.VMEM((SEG_PER_W, D), jnp.float32),
          pltpu.SemaphoreType.DMA,
          pltpu.SemaphoreType.DMA,
          pltpu.SemaphoreType.DMA,
          pltpu.SemaphoreType.DMA,
      ],
  )(emb_words, idx)


def _dot_t(x, w):
  return lax.dot_general(x, w, (((1,), (1,)), ((), ())),
                         preferred_element_type=jnp.float32)


def _mlp_body(x1_ref, x2_ref, w1a, w1b, b1, w2, b2, w3, b3, out_ref):
  h = _dot_t(x1_ref[...], w1a[...]) + _dot_t(x2_ref[...], w1b[...])
  h = jnp.maximum(h + b1[...], 0.0)
  h = jnp.maximum(_dot_t(h, w2[...]) + b2[...], 0.0)
  out_ref[...] = _dot_t(h, w3[...]) + b3[...]


def _mlp(q, w1a, w1b, b1, w2, b2, w3p, b3p):
  bb = 512
  grid = (BH // bb,)
  h1 = w1a.shape[0]
  h2 = w2.shape[0]
  return pl.pallas_call(
      _mlp_body,
      grid=grid,
      in_specs=[
          pl.BlockSpec((bb, D), lambda i: (i, 0)),
          pl.BlockSpec((bb, D), lambda i: (i + BH // bb, 0)),
          pl.BlockSpec((h1, D), lambda i: (0, 0)),
          pl.BlockSpec((h1, D), lambda i: (0, 0)),
          pl.BlockSpec((1, h1), lambda i: (0, 0)),
          pl.BlockSpec((h2, h1), lambda i: (0, 0)),
          pl.BlockSpec((1, h2), lambda i: (0, 0)),
          pl.BlockSpec((128, h2), lambda i: (0, 0)),
          pl.BlockSpec((1, 128), lambda i: (0, 0)),
      ],
      out_specs=pl.BlockSpec((bb, 128), lambda i: (i, 0)),
      out_shape=jax.ShapeDtypeStruct((BH, 128), jnp.float32),
  )(q, q, w1a, w1b, b1, w2, b2, w3p, b3p)


def kernel(x1, x2, emb, W1, b1, W2, b2, W3, b3):
  table = _pack_table(emb)

  inv_l = jnp.float32(1.0 / L)
  w1a = W1[:, :D] * inv_l
  w1b = W1[:, D:] * inv_l
  w2 = W2
  w3p = jnp.zeros((128, W2.shape[0]), jnp.float32).at[:2, :].set(W3)
  b3p = jnp.zeros((1, 128), jnp.float32).at[0, :2].set(b3)
  b1r = b1.reshape(1, -1)
  b2r = b2.reshape(1, -1)

  outs = []
  for h in range(NHALF):
    idx = jnp.concatenate([x1[h * BH:(h + 1) * BH].reshape(-1),
                           x2[h * BH:(h + 1) * BH].reshape(-1)])
    idx = idx.reshape(NW, NCHUNK, CHROWS)
    idx = jnp.pad(idx, ((0, 0), (0, 0), (0, CHPAD - CHROWS)))
    q = _sc_pool(table, idx)
    outs.append(_mlp(q, w1a, w1b, b1r, w2, b2r, w3p, b3p)[:, :2])

  return jnp.concatenate(outs, axis=0)


# speedup vs baseline: 2.0892x; 2.0728x over previous
import jax
import jax.numpy as jnp
from jax import lax
from jax.experimental import pallas as pl
from jax.experimental.pallas import tpu as pltpu
from jax.experimental.pallas import tpu_sc as plsc

B = 4096
L = 50
D = 128
VOCAB_ROWS = 100000
NC, NS = 2, 16
NW = NC * NS
NHALF = 1
BH = B // NHALF
SEG = 2 * BH
SEG_PER_W = SEG // NW
CH = 2
CHROWS = CH * L
CHPAD = 100
NCHUNK = SEG_PER_W // CH
DW = D // 2
NB = DW // 16
KBUF = 4


def _pack_body(x_ref, out_ref):
  xb = lax.bitcast_convert_type(x_ref[...], jnp.int32)
  rnd = lax.shift_right_logical(xb, 16) & 1
  bits = lax.shift_right_logical(xb + 0x7FFF + rnd, 16)
  lo = bits[:, :DW]
  hi = bits[:, DW:]
  out_ref[...] = lo | lax.shift_left(hi, 16)


def _pack_table(emb):
  rb = 2000
  return pl.pallas_call(
      _pack_body,
      grid=(VOCAB_ROWS // rb,),
      in_specs=[pl.BlockSpec((rb, D), lambda i: (i, 0))],
      out_specs=pl.BlockSpec((rb, DW), lambda i: (i, 0)),
      out_shape=jax.ShapeDtypeStruct((VOCAB_ROWS, DW), jnp.int32),
  )(emb)


def _sc_pool_body(emb_hbm, idx_hbm, out_hbm, idx_v, buf0, buf1, buf2, buf3,
                  res, sem0, sem1, sem2, sem3):
  c = lax.axis_index("c")
  s = lax.axis_index("s")
  wid = s * NC + c
  bufs = (buf0, buf1, buf2, buf3)
  sems = (sem0, sem1, sem2, sem3)

  pltpu.sync_copy(idx_hbm.at[wid], idx_v)

  hi_mask = jnp.full((16,), -65536, jnp.int32)

  def bf2x(words):
    a = lax.bitcast_convert_type(lax.shift_left(words, 16), jnp.float32)
    b = lax.bitcast_convert_type(lax.bitwise_and(words, hi_mask),
                                 jnp.float32)
    return a, b

  def reduce_chunk(buf, cbase):
    for sg in range(CH):
      acc_a = [None] * NB
      acc_b = [None] * NB
      for r in range(L):
        for d in range(NB):
          a, b = bf2x(buf[sg * L + r, pl.ds(16 * d, 16)])
          if r == 0:
            acc_a[d], acc_b[d] = a, b
          else:
            acc_a[d] = acc_a[d] + a
            acc_b[d] = acc_b[d] + b
      for d in range(NB):
        res[cbase + sg, pl.ds(16 * d, 16)] = acc_a[d]
        res[cbase + sg, pl.ds(DW + 16 * d, 16)] = acc_b[d]

  for j in range(KBUF):
    pltpu.async_copy(emb_hbm.at[idx_v.at[j]], bufs[j], sems[j])

  def body_ring(g, carry):
    c0 = KBUF * g
    for j in range(KBUF):
      cj = c0 + j
      pltpu.make_async_copy(emb_hbm.at[idx_v.at[cj]], bufs[j], sems[j]).wait()
      reduce_chunk(bufs[j], cj * CH)

      @pl.when(cj + KBUF < NCHUNK)
      def _():
        pltpu.async_copy(emb_hbm.at[idx_v.at[cj + KBUF]], bufs[j], sems[j])
    return carry

  lax.fori_loop(0, NCHUNK // KBUF, body_ring, 0)
  pltpu.sync_copy(res, out_hbm.at[pl.ds(wid * SEG_PER_W, SEG_PER_W)])


def _sc_pool(emb_words, idx):
  mesh = plsc.VectorSubcoreMesh(core_axis_name="c", subcore_axis_name="s")
  return pl.kernel(
      _sc_pool_body,
      out_type=jax.ShapeDtypeStruct((SEG, D), jnp.float32),
      mesh=mesh,
      compiler_params=pltpu.CompilerParams(use_tc_tiling_on_sc=False),
      scratch_types=[
          pltpu.VMEM((NCHUNK, CHPAD), jnp.int32),
          pltpu.VMEM((CHPAD, DW), jnp.int32),
          pltpu.VMEM((CHPAD, DW), jnp.int32),
          pltpu.VMEM((CHPAD, DW), jnp.int32),
          pltpu.VMEM((CHPAD, DW), jnp.int32),
          pltpu.VMEM((SEG_PER_W, D), jnp.float32),
          pltpu.SemaphoreType.DMA,
          pltpu.SemaphoreType.DMA,
          pltpu.SemaphoreType.DMA,
          pltpu.SemaphoreType.DMA,
      ],
  )(emb_words, idx)


def _dot_t(x, w):
  return lax.dot_general(x, w, (((1,), (1,)), ((), ())),
                         preferred_element_type=jnp.float32)


def _mlp_body(x1_ref, x2_ref, w1a, w1b, b1, w2, b2, w3, b3, out_ref):
  h = _dot_t(x1_ref[...], w1a[...]) + _dot_t(x2_ref[...], w1b[...])
  h = jnp.maximum(h + b1[...], 0.0)
  h = jnp.maximum(_dot_t(h, w2[...]) + b2[...], 0.0)
  out_ref[...] = _dot_t(h, w3[...]) + b3[...]


def _mlp(q, w1a, w1b, b1, w2, b2, w3p, b3p):
  bb = 512
  grid = (BH // bb,)
  h1 = w1a.shape[0]
  h2 = w2.shape[0]
  return pl.pallas_call(
      _mlp_body,
      grid=grid,
      in_specs=[
          pl.BlockSpec((bb, D), lambda i: (i, 0)),
          pl.BlockSpec((bb, D), lambda i: (i + BH // bb, 0)),
          pl.BlockSpec((h1, D), lambda i: (0, 0)),
          pl.BlockSpec((h1, D), lambda i: (0, 0)),
          pl.BlockSpec((1, h1), lambda i: (0, 0)),
          pl.BlockSpec((h2, h1), lambda i: (0, 0)),
          pl.BlockSpec((1, h2), lambda i: (0, 0)),
          pl.BlockSpec((128, h2), lambda i: (0, 0)),
          pl.BlockSpec((1, 128), lambda i: (0, 0)),
      ],
      out_specs=pl.BlockSpec((bb, 128), lambda i: (i, 0)),
      out_shape=jax.ShapeDtypeStruct((BH, 128), jnp.float32),
  )(q, q, w1a, w1b, b1, w2, b2, w3p, b3p)


def kernel(x1, x2, emb, W1, b1, W2, b2, W3, b3):
  table = _pack_table(emb)

  inv_l = jnp.float32(1.0 / L)
  w1a = W1[:, :D] * inv_l
  w1b = W1[:, D:] * inv_l
  w2 = W2
  w3p = jnp.zeros((128, W2.shape[0]), jnp.float32).at[:2, :].set(W3)
  b3p = jnp.zeros((1, 128), jnp.float32).at[0, :2].set(b3)
  b1r = b1.reshape(1, -1)
  b2r = b2.reshape(1, -1)

  outs = []
  for h in range(NHALF):
    idx = jnp.concatenate([x1[h * BH:(h + 1) * BH].reshape(-1),
                           x2[h * BH:(h + 1) * BH].reshape(-1)])
    idx = idx.reshape(NW, NCHUNK, CHROWS)
    idx = jnp.pad(idx, ((0, 0), (0, 0), (0, CHPAD - CHROWS)))
    q = _sc_pool(table, idx)
    outs.append(_mlp(q, w1a, w1b, b1r, w2, b2r, w3p, b3p)[:, :2])

  return jnp.concatenate(outs, axis=0)
